# baseline (device time: 140116 ns/iter reference)
import jax
import jax.numpy as jnp
from jax import lax
from jax.experimental import pallas as pl
from jax.experimental.pallas import tpu as pltpu

N_DEV = 16
B, SQ, DM = 2, 512, 768
DH = 64
BLK = 64
ROWS = B * SQ
CH = ROWS // N_DEV


def kernel(x, Wq, K_ext, V_ext, Wo):
    H = K_ext.shape[2]
    HD = H * DH

    idx = lax.axis_index("i")
    x2 = x.reshape(ROWS, DM).astype(jnp.bfloat16)
    wq_s = lax.dynamic_slice(Wq, (0, idx * HD), (DM, HD)).astype(jnp.bfloat16)
    wo_s = lax.dynamic_slice(Wo, (idx * HD, 0), (HD, DM)).astype(jnp.bfloat16)
    k = jnp.transpose(K_ext, (0, 2, 1, 3)).astype(jnp.bfloat16)
    v = jnp.transpose(V_ext, (0, 2, 1, 3)).astype(jnp.bfloat16)

    def body(x_ref, wq_ref, k_ref, v_ref, wo_ref, o_ref,
             q_ref, ctx_ref, part_ref, rs_send, rs_recv,
             rs_ssem, rs_rsem, ag_ssem, ag_rsem):
        me = lax.axis_index("i")
        left = (me - 1) % N_DEV
        right = (me + 1) % N_DEV

        q_ref[...] = jnp.dot(
            x_ref[...], wq_ref[...], preferred_element_type=jnp.float32
        ).astype(jnp.bfloat16)

        qb = lax.broadcasted_iota(jnp.int32, (SQ, SQ), 0) // BLK
        kb = lax.broadcasted_iota(jnp.int32, (SQ, SQ), 1) // BLK
        mask = (qb == kb) | (kb == 0) | ((qb + kb) % 3 == 0)
        bias = jnp.where(mask, 0.0, -1e9).astype(jnp.float32)

        for b in range(B):
            for h in range(H):
                q = q_ref[b * SQ:(b + 1) * SQ, h * DH:(h + 1) * DH]
                s = lax.dot_general(
                    q, k_ref[b, h], (((1,), (1,)), ((), ())),
                    preferred_element_type=jnp.float32,
                )
                s = s * 0.125 + bias
                m = jnp.max(s, axis=1, keepdims=True)
                e = jnp.exp(s - m)
                p = (e / jnp.sum(e, axis=1, keepdims=True)).astype(jnp.bfloat16)
                ctx = jnp.dot(p, v_ref[b, h], preferred_element_type=jnp.float32)
                ctx_ref[b * SQ:(b + 1) * SQ, h * DH:(h + 1) * DH] = (
                    ctx.astype(jnp.bfloat16))

        part_ref[...] = jnp.dot(
            ctx_ref[...], wo_ref[...], preferred_element_type=jnp.float32)

        bar = pltpu.get_barrier_semaphore()
        pl.semaphore_signal(bar, inc=1, device_id=(left,),
                            device_id_type=pl.DeviceIdType.MESH)
        pl.semaphore_signal(bar, inc=1, device_id=(right,),
                            device_id_type=pl.DeviceIdType.MESH)
        pl.semaphore_wait(bar, 2)

        for hp in range(N_DEV - 1):
            sidx = (me - hp) % N_DEV
            if hp == 0:
                src = part_ref.at[pl.ds(sidx * CH, CH), :]
            else:
                rs_send[hp] = rs_recv[hp - 1] + part_ref[pl.ds(sidx * CH, CH), :]
                src = rs_send.at[hp]
            rdma = pltpu.make_async_remote_copy(
                src_ref=src, dst_ref=rs_recv.at[hp],
                send_sem=rs_ssem.at[hp % 4], recv_sem=rs_rsem.at[hp % 4],
                device_id=(right,), device_id_type=pl.DeviceIdType.MESH)
            rdma.start()
            rdma.wait()

        own = (me + 1) % N_DEV
        o_ref[pl.ds(own * CH, CH), :] = (
            rs_recv[N_DEV - 2] + part_ref[pl.ds(own * CH, CH), :])

        for g in range(N_DEV - 1):
            aidx = (me + 1 - g) % N_DEV
            rdma = pltpu.make_async_remote_copy(
                src_ref=o_ref.at[pl.ds(aidx * CH, CH), :],
                dst_ref=o_ref.at[pl.ds(aidx * CH, CH), :],
                send_sem=ag_ssem.at[g % 4], recv_sem=ag_rsem.at[g % 4],
                device_id=(right,), device_id_type=pl.DeviceIdType.MESH)
            rdma.start()
            rdma.wait()

    out = pl.pallas_call(
        body,
        out_shape=jax.ShapeDtypeStruct((ROWS, DM), jnp.float32),
        in_specs=[pl.BlockSpec(memory_space=pltpu.VMEM)] * 5,
        out_specs=pl.BlockSpec(memory_space=pltpu.VMEM),
        scratch_shapes=[
            pltpu.VMEM((ROWS, HD), jnp.bfloat16),
            pltpu.VMEM((ROWS, HD), jnp.bfloat16),
            pltpu.VMEM((ROWS, DM), jnp.float32),
            pltpu.VMEM((N_DEV - 1, CH, DM), jnp.float32),
            pltpu.VMEM((N_DEV - 1, CH, DM), jnp.float32),
            pltpu.SemaphoreType.DMA((4,)),
            pltpu.SemaphoreType.DMA((4,)),
            pltpu.SemaphoreType.DMA((4,)),
            pltpu.SemaphoreType.DMA((4,)),
        ],
        compiler_params=pltpu.CompilerParams(collective_id=0),
    )(x2, wq_s, k, v, wo_s)

    return out.reshape(B, SQ, DM)


# device time: 56396 ns/iter; 2.4845x vs baseline; 2.4845x over previous
import jax
import jax.numpy as jnp
from jax import lax
from jax.experimental import pallas as pl
from jax.experimental.pallas import tpu as pltpu

N_DEV = 16
B, SQ, DM = 2, 512, 768
DH = 64
BLK = 64
ROWS = B * SQ
CH = ROWS // N_DEV


def kernel(x, Wq, K_ext, V_ext, Wo):
    H = K_ext.shape[2]
    HD = H * DH

    idx = lax.axis_index("i")
    x2 = x.reshape(ROWS, DM).astype(jnp.bfloat16)
    wq_s = lax.dynamic_slice(Wq, (0, idx * HD), (DM, HD)).astype(jnp.bfloat16)
    wo_s = lax.dynamic_slice(Wo, (idx * HD, 0), (HD, DM)).astype(jnp.bfloat16)
    k = jnp.transpose(K_ext, (0, 2, 1, 3)).astype(jnp.bfloat16)
    v = jnp.transpose(V_ext, (0, 2, 1, 3)).astype(jnp.bfloat16)

    def body(x_ref, wq_ref, k_ref, v_ref, wo_ref, o_ref,
             q_ref, ctx_ref, part_ref, red_ref, rs_recv,
             rs_ssem, rs_rsem, ag_ssem, ag_rsem):
        me = lax.axis_index("i")

        q_ref[...] = jnp.dot(
            x_ref[...], wq_ref[...], preferred_element_type=jnp.float32
        ).astype(jnp.bfloat16)

        qb = lax.broadcasted_iota(jnp.int32, (SQ, SQ), 0) // BLK
        kb = lax.broadcasted_iota(jnp.int32, (SQ, SQ), 1) // BLK
        mask = (qb == kb) | (kb == 0) | ((qb + kb) % 3 == 0)
        bias = jnp.where(mask, 0.0, -1e9).astype(jnp.float32)

        for b in range(B):
            for h in range(H):
                q = q_ref[b * SQ:(b + 1) * SQ, h * DH:(h + 1) * DH]
                s = lax.dot_general(
                    q, k_ref[b, h], (((1,), (1,)), ((), ())),
                    preferred_element_type=jnp.float32,
                )
                s = s * 0.125 + bias
                m = jnp.max(s, axis=1, keepdims=True)
                e = jnp.exp(s - m)
                p = (e / jnp.sum(e, axis=1, keepdims=True)).astype(jnp.bfloat16)
                ctx = jnp.dot(p, v_ref[b, h], preferred_element_type=jnp.float32)
                ctx_ref[b * SQ:(b + 1) * SQ, h * DH:(h + 1) * DH] = (
                    ctx.astype(jnp.bfloat16))

        part_ref[...] = jnp.dot(
            ctx_ref[...], wo_ref[...], preferred_element_type=jnp.float32
        ).astype(jnp.bfloat16)

        bar = pltpu.get_barrier_semaphore()
        for j in range(N_DEV):
            pl.semaphore_signal(bar, inc=1, device_id=(j,),
                                device_id_type=pl.DeviceIdType.MESH)
        pl.semaphore_wait(bar, N_DEV)

        rs_descs = []
        for j in range(N_DEV):
            d = pltpu.make_async_remote_copy(
                src_ref=part_ref.at[pl.ds(j * CH, CH), :],
                dst_ref=rs_recv.at[me],
                send_sem=rs_ssem.at[j],
                recv_sem=rs_rsem,
                device_id=(j,), device_id_type=pl.DeviceIdType.MESH)
            d.start()
            rs_descs.append(d)
        for d in rs_descs:
            d.wait_recv()

        acc = rs_recv[0].astype(jnp.float32)
        for s_ in range(1, N_DEV):
            acc = acc + rs_recv[s_].astype(jnp.float32)
        red_ref[...] = acc.astype(jnp.bfloat16)

        ag_descs = []
        for j in range(N_DEV):
            d = pltpu.make_async_remote_copy(
                src_ref=red_ref,
                dst_ref=o_ref.at[pl.ds(me * CH, CH), :],
                send_sem=ag_ssem.at[j],
                recv_sem=ag_rsem,
                device_id=(j,), device_id_type=pl.DeviceIdType.MESH)
            d.start()
            ag_descs.append(d)
        for d in ag_descs:
            d.wait_recv()

        for d in rs_descs:
            d.wait_send()
        for d in ag_descs:
            d.wait_send()

    out = pl.pallas_call(
        body,
        out_shape=jax.ShapeDtypeStruct((ROWS, DM), jnp.bfloat16),
        in_specs=[pl.BlockSpec(memory_space=pltpu.VMEM)] * 5,
        out_specs=pl.BlockSpec(memory_space=pltpu.VMEM),
        scratch_shapes=[
            pltpu.VMEM((ROWS, HD), jnp.bfloat16),
            pltpu.VMEM((ROWS, HD), jnp.bfloat16),
            pltpu.VMEM((ROWS, DM), jnp.bfloat16),
            pltpu.VMEM((CH, DM), jnp.bfloat16),
            pltpu.VMEM((N_DEV, CH, DM), jnp.bfloat16),
            pltpu.SemaphoreType.DMA((N_DEV,)),
            pltpu.SemaphoreType.DMA,
            pltpu.SemaphoreType.DMA((N_DEV,)),
            pltpu.SemaphoreType.DMA,
        ],
        compiler_params=pltpu.CompilerParams(collective_id=0),
    )(x2, wq_s, k, v, wo_s)

    return out.astype(jnp.float32).reshape(B, SQ, DM)
